# baseline (speedup 1.0000x reference)
import functools

import jax
import jax.numpy as jnp
from jax import lax
from jax.experimental import pallas as pl
from jax.experimental.pallas import tpu as pltpu
from jax.experimental.pallas import tpu_sc as plsc

N = 10000
E = 320000
M = 3
IN_DIM = 128
K = 8
OUT = 16
HID = 128

NC = 2
NS = 16
NW = NC * NS
EPW = E // NW
C = 40
NCH = EPW // C

BLK = 2000
NB = N // BLK


def _dense_body(h_ref, w_ref, al_ref, ar_ref, ft_ref, el_ref, er_ref):
    hb = h_ref[...]
    fm = jnp.dot(hb, w_ref[0], preferred_element_type=jnp.float32,
                 precision=lax.Precision.HIGHEST)
    el = jnp.dot(fm, al_ref[0], preferred_element_type=jnp.float32,
                 precision=lax.Precision.HIGHEST)
    er = jnp.dot(fm, ar_ref[0], preferred_element_type=jnp.float32,
                 precision=lax.Precision.HIGHEST)
    pad = jnp.zeros((hb.shape[0], 128 - 2 * K), jnp.float32)
    ft_ref[0] = fm
    el_ref[0] = jnp.concatenate([el, el, pad], axis=1)
    er_ref[0] = jnp.concatenate([er, er, pad], axis=1)


def _dense_stage(h, W, Al, Ar):
    return pl.pallas_call(
        _dense_body,
        grid=(M, NB),
        in_specs=[
            pl.BlockSpec((BLK, IN_DIM), lambda m, i: (i, 0)),
            pl.BlockSpec((1, IN_DIM, K * OUT), lambda m, i: (m, 0, 0)),
            pl.BlockSpec((1, K * OUT, K), lambda m, i: (m, 0, 0)),
            pl.BlockSpec((1, K * OUT, K), lambda m, i: (m, 0, 0)),
        ],
        out_specs=[
            pl.BlockSpec((1, BLK, K * OUT), lambda m, i: (m, i, 0)),
            pl.BlockSpec((1, BLK, 128), lambda m, i: (m, i, 0)),
            pl.BlockSpec((1, BLK, 128), lambda m, i: (m, i, 0)),
        ],
        out_shape=[
            jax.ShapeDtypeStruct((M, N, K * OUT), jnp.float32),
            jax.ShapeDtypeStruct((M, N, 128), jnp.float32),
            jax.ShapeDtypeStruct((M, N, 128), jnp.float32),
        ],
    )(h, W, Al, Ar)


def _sc_body(ft0, ft1, ft2, eld0, eld1, eld2, erd0, erd1, erd2,
             src0, dst0, src1, dst1, src2, dst2,
             outP, denP,
             src_c, dst_c, dst_cx, den_idx, ftg, elg, erg, msgb2,
             out_acc, den_acc, sem):
    ci = lax.axis_index("c")
    si = lax.axis_index("s")
    wid = si * NC + ci
    fts = [ft0, ft1, ft2]
    elds = [eld0, eld1, eld2]
    erds = [erd0, erd1, erd2]
    srcs = [src0, src1, src2]
    dsts = [dst0, dst1, dst2]

    i16 = lax.iota(jnp.int32, 16)
    z16 = jnp.zeros((16,), jnp.float32)

    def fill_seq(ref, base):
        ref[0, pl.ds(0, 16)] = i16 + base
        ref[0, pl.ds(16, 16)] = i16 + (base + 16)
        ref[0, pl.ds(24, 16)] = i16 + (base + 24)

    def _out_rounds(fn):
        @pl.when(si < 15)
        def _():
            for r in range(16):
                fn(si * 624, r * C if r < 15 else 584)

        @pl.when(si == 15)
        def _():
            for r in range(16):
                fn(15 * 624, r * C)

    def _den_rounds(fn):
        for r in range(2):
            fn(si * 80, r * C)

    for m in range(M):
        ft_t = fts[m]
        eld_t = elds[m]
        erd_t = erds[m]
        src_t = srcs[m]
        dst_t = dsts[m]

        def zfill_body(t, _):
            for k in range(K):
                msgb2[t, pl.ds(16 * k, 16)] = z16
            return 0

        lax.fori_loop(0, C, zfill_body, 0)

        def _zero_out(base, off):
            fill_seq(dst_c, base + off)
            pltpu.sync_copy(msgb2, out_acc.at[dst_c.at[0]])

        def _zero_den(base, off):
            fill_seq(den_idx, base + off)
            pltpu.sync_copy(msgb2, den_acc.at[den_idx.at[0]])

        _out_rounds(_zero_out)
        _den_rounds(_zero_den)
        plsc.subcore_barrier()

        def chunk_body(j, _):
            base = wid * EPW + j * C
            pltpu.sync_copy(src_t.at[pl.ds(base, C)], src_c)
            pltpu.sync_copy(dst_t.at[pl.ds(base, C)], dst_c.at[0])
            pltpu.sync_copy(dst_t.at[pl.ds(base, C)], dst_cx.at[pl.ds(0, C)])
            a1 = pltpu.async_copy(ft_t.at[src_c], ftg, sem)
            a2 = pltpu.async_copy(eld_t.at[src_c], elg, sem)
            a3 = pltpu.async_copy(erd_t.at[dst_c.at[0]], erg, sem)
            den_idx[0, pl.ds(0, 16)] = lax.shift_right_logical(
                dst_cx[pl.ds(0, 16)], 3)
            den_idx[0, pl.ds(16, 16)] = lax.shift_right_logical(
                dst_cx[pl.ds(16, 16)], 3)
            den_idx[0, pl.ds(24, 16)] = lax.shift_right_logical(
                dst_cx[pl.ds(24, 16)], 3)
            a1.wait()
            a2.wait()
            a3.wait()

            def edge_group(u, _):
                dvec = dst_cx[pl.ds(8 * u, 16)]
                for jj in range(8):
                    t8 = 8 * u + jj
                    q = dvec[jj]
                    v = elg[t8, pl.ds(0, 16)] + erg[t8, pl.ds(0, 16)]
                    s16 = jnp.exp(jnp.maximum(v, 0.2 * v))
                    for k0 in range(K):
                        msgb2[t8, pl.ds(16 * k0, 16)] = z16
                    msgb2[t8, pl.ds((q % 8) * 16, 16)] = s16
                    for k in range(K):
                        ftg[t8, pl.ds(16 * k, 16)] = (
                            ftg[t8, pl.ds(16 * k, 16)] * s16[k])
                return 0

            lax.fori_loop(0, C // 8, edge_group, 0)

            pltpu.sync_copy(msgb2, den_acc.at[den_idx.at[0]], add=True)
            pltpu.sync_copy(ftg, out_acc.at[dst_c.at[0]], add=True)
            return 0

        lax.fori_loop(0, NCH, chunk_body, 0)
        plsc.subcore_barrier()

        def _flush_out(base, off):
            fill_seq(dst_c, base + off)
            pltpu.sync_copy(out_acc.at[dst_c.at[0]], msgb2)
            for c in range(NC):
                @pl.when(ci == c)
                def _(c=c, m=m):
                    pltpu.sync_copy(msgb2, outP.at[m, c, pl.ds(base + off, C)])

        def _flush_den(base, off):
            fill_seq(den_idx, base + off)
            pltpu.sync_copy(den_acc.at[den_idx.at[0]], msgb2)
            for c in range(NC):
                @pl.when(ci == c)
                def _(c=c, m=m):
                    pltpu.sync_copy(msgb2, denP.at[m, c, pl.ds(base + off, C)])

        _out_rounds(_flush_out)
        _den_rounds(_flush_den)
        plsc.subcore_barrier()


_sc_edge_stage = functools.partial(
    pl.kernel,
    out_type=[
        jax.ShapeDtypeStruct((M, NC, N, K * OUT), jnp.float32),
        jax.ShapeDtypeStruct((M, NC, 1280, 128), jnp.float32),
    ],
    mesh=plsc.VectorSubcoreMesh(core_axis_name="c", subcore_axis_name="s",
                                num_cores=NC, num_subcores=NS),
    compiler_params=pltpu.CompilerParams(needs_layout_passes=False),
    scratch_types=[
        pltpu.VMEM((C,), jnp.int32),
        pltpu.VMEM((1, C), jnp.int32),
        pltpu.VMEM((C + 8,), jnp.int32),
        pltpu.VMEM((1, C), jnp.int32),
        pltpu.VMEM((C, K * OUT), jnp.float32),
        pltpu.VMEM((C, 128), jnp.float32),
        pltpu.VMEM((C, 128), jnp.float32),
        pltpu.VMEM((C, 128), jnp.float32),
        pltpu.VMEM_SHARED((N, K * OUT), jnp.float32),
        pltpu.VMEM_SHARED((1280, 128), jnp.float32),
        pltpu.SemaphoreType.DMA,
    ],
)(_sc_body)


def _combine_body(outP_ref, denP_ref, bias_ref, w1_ref, b1_ref, w2_ref,
                  z_ref, wp_ref):
    rowi = lax.broadcasted_iota(jnp.int32, (K, K * OUT), 0)
    coli = lax.broadcasted_iota(jnp.int32, (K, K * OUT), 1)
    P = (coli // OUT == rowi).astype(jnp.float32)
    wp = jnp.zeros((8, 128), jnp.float32)
    mrow = lax.broadcasted_iota(jnp.int32, (8, 128), 0)
    mcol = lax.broadcasted_iota(jnp.int32, (8, 128), 1)
    for m in range(M):
        o = outP_ref[m, 0]
        d = denP_ref[m, 0][:, :K]
        for c in range(1, NC):
            o = o + outP_ref[m, c]
            d = d + denP_ref[m, c][:, :K]
        dx = jnp.dot(d, P, preferred_element_type=jnp.float32,
                     precision=lax.Precision.HIGHEST)
        z = o / jnp.maximum(dx, 1e-30) + bias_ref[m][None, :]
        z = jnp.where(z > 0, z, jnp.exp(jnp.minimum(z, 0.0)) - 1.0)
        z_ref[m] = z
        t = jnp.tanh(jnp.dot(z, w1_ref[...], preferred_element_type=jnp.float32,
                             precision=lax.Precision.HIGHEST) + b1_ref[...][None, :])
        wcol = jnp.dot(t, w2_ref[...], preferred_element_type=jnp.float32,
                       precision=lax.Precision.HIGHEST)
        wsum = jnp.sum(wcol)
        wp = wp + wsum * ((mrow == 0) & (mcol == m)).astype(jnp.float32)
    wp_ref[0] = wp


def _combine_stage(outP, denP, bias, sem_W1, sem_b1, sem_W2):
    return pl.pallas_call(
        _combine_body,
        grid=(NB,),
        in_specs=[
            pl.BlockSpec((M, NC, BLK, K * OUT), lambda i: (0, 0, i, 0)),
            pl.BlockSpec((M, NC, BLK, 2 * K), lambda i: (0, 0, i, 0)),
            pl.BlockSpec((M, K * OUT), lambda i: (0, 0)),
            pl.BlockSpec((K * OUT, HID), lambda i: (0, 0)),
            pl.BlockSpec((HID,), lambda i: (0,)),
            pl.BlockSpec((HID, 1), lambda i: (0, 0)),
        ],
        out_specs=[
            pl.BlockSpec((M, BLK, K * OUT), lambda i: (0, i, 0)),
            pl.BlockSpec((1, 8, 128), lambda i: (i, 0, 0)),
        ],
        out_shape=[
            jax.ShapeDtypeStruct((M, N, K * OUT), jnp.float32),
            jax.ShapeDtypeStruct((NB, 8, 128), jnp.float32),
        ],
    )(outP, denP, bias, sem_W1, sem_b1, sem_W2)


def _final_body(z_ref, beta_ref, out_ref):
    acc = z_ref[0] * beta_ref[0:1, 0:1]
    acc = acc + z_ref[1] * beta_ref[0:1, 1:2]
    acc = acc + z_ref[2] * beta_ref[0:1, 2:3]
    out_ref[...] = acc


def _final_stage(z, beta_pad):
    return pl.pallas_call(
        _final_body,
        grid=(NB,),
        in_specs=[
            pl.BlockSpec((M, BLK, K * OUT), lambda i: (0, i, 0)),
            pl.BlockSpec((8, 128), lambda i: (0, 0)),
        ],
        out_specs=pl.BlockSpec((BLK, K * OUT), lambda i: (i, 0)),
        out_shape=jax.ShapeDtypeStruct((N, K * OUT), jnp.float32),
    )(z, beta_pad)


def kernel(h, edge_index_0, edge_index_1, edge_index_2, W, attn_l, attn_r,
           bias, sem_W1, sem_b1, sem_W2):
    rows = jnp.arange(K * OUT)
    Al = jnp.zeros((M, K * OUT, K), jnp.float32).at[
        :, rows, rows // OUT].set(attn_l.reshape(M, K * OUT))
    Ar = jnp.zeros((M, K * OUT, K), jnp.float32).at[
        :, rows, rows // OUT].set(attn_r.reshape(M, K * OUT))

    ft3, el3, er3 = _dense_stage(h, W, Al, Ar)

    outP, denPp = _sc_edge_stage(
        ft3[0], ft3[1], ft3[2], el3[0], el3[1], el3[2], er3[0], er3[1], er3[2],
        edge_index_0[0], edge_index_0[1], edge_index_1[0], edge_index_1[1],
        edge_index_2[0], edge_index_2[1])
    denP = denPp.reshape(M, NC, 1280 * 8, 16)[:, :, :N, :]

    z, wp = _combine_stage(outP, denP, bias, sem_W1, sem_b1, sem_W2)
    wsum = wp[:, 0, :M].sum(axis=0) / N
    beta = jax.nn.softmax(wsum)
    beta_pad = jnp.zeros((8, 128), jnp.float32).at[0, :M].set(beta)
    return _final_stage(z, beta_pad)


# speedup vs baseline: 1.2750x; 1.2750x over previous
import functools

import jax
import jax.numpy as jnp
from jax import lax
from jax.experimental import pallas as pl
from jax.experimental.pallas import tpu as pltpu
from jax.experimental.pallas import tpu_sc as plsc

N = 10000
E = 320000
M = 3
IN_DIM = 128
K = 8
OUT = 16
HID = 128

NC = 2
NS = 16
NW = NC * NS
EPW = E // NW
C = 40
NCH = EPW // C

BLK = 2000
NB = N // BLK


def _dense_body(h_ref, w_ref, al_ref, ar_ref, ft_ref, el_ref, er_ref):
    hb = h_ref[...]
    fm = jnp.dot(hb, w_ref[0], preferred_element_type=jnp.float32,
                 precision=lax.Precision.HIGHEST)
    el = jnp.dot(fm, al_ref[0], preferred_element_type=jnp.float32,
                 precision=lax.Precision.HIGHEST)
    er = jnp.dot(fm, ar_ref[0], preferred_element_type=jnp.float32,
                 precision=lax.Precision.HIGHEST)
    pad = jnp.zeros((hb.shape[0], 128 - 2 * K), jnp.float32)
    ft_ref[0] = fm
    el_ref[0] = jnp.concatenate([el, el, pad], axis=1)
    er_ref[0] = jnp.concatenate([er, er, pad], axis=1)


def _dense_stage(h, W, Al, Ar):
    return pl.pallas_call(
        _dense_body,
        grid=(M, NB),
        in_specs=[
            pl.BlockSpec((BLK, IN_DIM), lambda m, i: (i, 0)),
            pl.BlockSpec((1, IN_DIM, K * OUT), lambda m, i: (m, 0, 0)),
            pl.BlockSpec((1, K * OUT, K), lambda m, i: (m, 0, 0)),
            pl.BlockSpec((1, K * OUT, K), lambda m, i: (m, 0, 0)),
        ],
        out_specs=[
            pl.BlockSpec((1, BLK, K * OUT), lambda m, i: (m, i, 0)),
            pl.BlockSpec((1, BLK, 128), lambda m, i: (m, i, 0)),
            pl.BlockSpec((1, BLK, 128), lambda m, i: (m, i, 0)),
        ],
        out_shape=[
            jax.ShapeDtypeStruct((M, N, K * OUT), jnp.float32),
            jax.ShapeDtypeStruct((M, N, 128), jnp.float32),
            jax.ShapeDtypeStruct((M, N, 128), jnp.float32),
        ],
    )(h, W, Al, Ar)


def _sc_body(ft0, ft1, ft2, eld0, eld1, eld2, erd0, erd1, erd2,
             src0, dst0, src1, dst1, src2, dst2,
             outP, denP,
             src_c, dst_c, dst_cx, den_idx, ftg, elg, erg, msgb2,
             out_acc, den_acc, sem):
    ci = lax.axis_index("c")
    si = lax.axis_index("s")
    wid = si * NC + ci
    fts = [ft0, ft1, ft2]
    elds = [eld0, eld1, eld2]
    erds = [erd0, erd1, erd2]
    srcs = [src0, src1, src2]
    dsts = [dst0, dst1, dst2]

    i16 = lax.iota(jnp.int32, 16)
    z16 = jnp.zeros((16,), jnp.float32)

    def fill_seq(ref, base):
        ref[0, pl.ds(0, 16)] = i16 + base
        ref[0, pl.ds(16, 16)] = i16 + (base + 16)
        ref[0, pl.ds(24, 16)] = i16 + (base + 24)

    def _out_rounds(fn):
        @pl.when(si < 15)
        def _():
            for r in range(16):
                fn(si * 624, r * C if r < 15 else 584)

        @pl.when(si == 15)
        def _():
            for r in range(16):
                fn(15 * 624, r * C)

    def _den_rounds(fn):
        for r in range(2):
            fn(si * 80, r * C)

    for m in range(M):
        ft_t = fts[m]
        eld_t = elds[m]
        erd_t = erds[m]
        src_t = srcs[m]
        dst_t = dsts[m]

        def zfill_body(t, _):
            for k in range(K):
                msgb2[t, pl.ds(16 * k, 16)] = z16
            return 0

        lax.fori_loop(0, C, zfill_body, 0)

        def _zero_out(base, off):
            fill_seq(dst_c, base + off)
            pltpu.sync_copy(msgb2, out_acc.at[dst_c.at[0]])

        def _zero_den(base, off):
            fill_seq(den_idx, base + off)
            pltpu.sync_copy(msgb2, den_acc.at[den_idx.at[0]])

        _out_rounds(_zero_out)
        _den_rounds(_zero_den)
        plsc.subcore_barrier()

        def chunk_body(j, _):
            base = wid * EPW + j * C
            ai = pltpu.async_copy(src_t.at[pl.ds(base, C)], src_c, sem)
            aj = pltpu.async_copy(dst_t.at[pl.ds(base, C)],
                                  dst_cx.at[pl.ds(0, C)], sem)
            ai.wait()
            aj.wait()
            a1 = pltpu.async_copy(ft_t.at[src_c], ftg, sem)
            a2 = pltpu.async_copy(eld_t.at[src_c], elg, sem)
            a3 = pltpu.async_copy(erd_t.at[dst_cx.at[pl.ds(0, C)]], erg, sem)
            for g in range(3):
                o = (0, 16, 24)[g]
                dv = dst_cx[pl.ds(o, 16)]
                dst_c[0, pl.ds(o, 16)] = dv
                den_idx[0, pl.ds(o, 16)] = lax.shift_right_logical(dv, 3)
            a1.wait()
            a2.wait()
            a3.wait()

            def edge_group(u, _):
                dvec = dst_cx[pl.ds(8 * u, 16)]
                for jj in range(8):
                    t8 = 8 * u + jj
                    q = dvec[jj]
                    v = elg[t8, pl.ds(0, 16)] + erg[t8, pl.ds(0, 16)]
                    s16 = jnp.exp(jnp.maximum(v, 0.2 * v))
                    for k0 in range(K):
                        msgb2[t8, pl.ds(16 * k0, 16)] = z16
                    msgb2[t8, pl.ds((q % 8) * 16, 16)] = s16
                    for k in range(K):
                        ftg[t8, pl.ds(16 * k, 16)] = (
                            ftg[t8, pl.ds(16 * k, 16)] * s16[k])
                return 0

            lax.fori_loop(0, C // 8, edge_group, 0)

            s1 = pltpu.async_copy(msgb2, den_acc.at[den_idx.at[0]], sem,
                                  add=True)
            s2 = pltpu.async_copy(ftg, out_acc.at[dst_c.at[0]], sem, add=True)
            s1.wait()
            s2.wait()
            return 0

        lax.fori_loop(0, NCH, chunk_body, 0)
        plsc.subcore_barrier()

        def _flush_out(base, off):
            fill_seq(dst_c, base + off)
            pltpu.sync_copy(out_acc.at[dst_c.at[0]], msgb2)
            for c in range(NC):
                @pl.when(ci == c)
                def _(c=c, m=m):
                    pltpu.sync_copy(msgb2, outP.at[m, c, pl.ds(base + off, C)])

        def _flush_den(base, off):
            fill_seq(den_idx, base + off)
            pltpu.sync_copy(den_acc.at[den_idx.at[0]], msgb2)
            for c in range(NC):
                @pl.when(ci == c)
                def _(c=c, m=m):
                    pltpu.sync_copy(msgb2, denP.at[m, c, pl.ds(base + off, C)])

        _out_rounds(_flush_out)
        _den_rounds(_flush_den)
        plsc.subcore_barrier()


_sc_edge_stage = functools.partial(
    pl.kernel,
    out_type=[
        jax.ShapeDtypeStruct((M, NC, N, K * OUT), jnp.float32),
        jax.ShapeDtypeStruct((M, NC, 1280, 128), jnp.float32),
    ],
    mesh=plsc.VectorSubcoreMesh(core_axis_name="c", subcore_axis_name="s",
                                num_cores=NC, num_subcores=NS),
    compiler_params=pltpu.CompilerParams(needs_layout_passes=False),
    scratch_types=[
        pltpu.VMEM((C,), jnp.int32),
        pltpu.VMEM((1, C), jnp.int32),
        pltpu.VMEM((C + 8,), jnp.int32),
        pltpu.VMEM((1, C), jnp.int32),
        pltpu.VMEM((C, K * OUT), jnp.float32),
        pltpu.VMEM((C, 128), jnp.float32),
        pltpu.VMEM((C, 128), jnp.float32),
        pltpu.VMEM((C, 128), jnp.float32),
        pltpu.VMEM_SHARED((N, K * OUT), jnp.float32),
        pltpu.VMEM_SHARED((1280, 128), jnp.float32),
        pltpu.SemaphoreType.DMA,
    ],
)(_sc_body)


def _combine_body(outP_ref, denP_ref, bias_ref, w1_ref, b1_ref, w2_ref,
                  z_ref, wp_ref):
    rowi = lax.broadcasted_iota(jnp.int32, (K, K * OUT), 0)
    coli = lax.broadcasted_iota(jnp.int32, (K, K * OUT), 1)
    P = (coli // OUT == rowi).astype(jnp.float32)
    wp = jnp.zeros((8, 128), jnp.float32)
    mrow = lax.broadcasted_iota(jnp.int32, (8, 128), 0)
    mcol = lax.broadcasted_iota(jnp.int32, (8, 128), 1)
    for m in range(M):
        o = outP_ref[m, 0]
        d = denP_ref[m, 0][:, :K]
        for c in range(1, NC):
            o = o + outP_ref[m, c]
            d = d + denP_ref[m, c][:, :K]
        dx = jnp.dot(d, P, preferred_element_type=jnp.float32,
                     precision=lax.Precision.HIGHEST)
        z = o / jnp.maximum(dx, 1e-30) + bias_ref[m][None, :]
        z = jnp.where(z > 0, z, jnp.exp(jnp.minimum(z, 0.0)) - 1.0)
        z_ref[m] = z
        t = jnp.tanh(jnp.dot(z, w1_ref[...], preferred_element_type=jnp.float32,
                             precision=lax.Precision.HIGHEST) + b1_ref[...][None, :])
        wcol = jnp.dot(t, w2_ref[...], preferred_element_type=jnp.float32,
                       precision=lax.Precision.HIGHEST)
        wsum = jnp.sum(wcol)
        wp = wp + wsum * ((mrow == 0) & (mcol == m)).astype(jnp.float32)
    wp_ref[0] = wp


def _combine_stage(outP, denP, bias, sem_W1, sem_b1, sem_W2):
    return pl.pallas_call(
        _combine_body,
        grid=(NB,),
        in_specs=[
            pl.BlockSpec((M, NC, BLK, K * OUT), lambda i: (0, 0, i, 0)),
            pl.BlockSpec((M, NC, BLK, 2 * K), lambda i: (0, 0, i, 0)),
            pl.BlockSpec((M, K * OUT), lambda i: (0, 0)),
            pl.BlockSpec((K * OUT, HID), lambda i: (0, 0)),
            pl.BlockSpec((HID,), lambda i: (0,)),
            pl.BlockSpec((HID, 1), lambda i: (0, 0)),
        ],
        out_specs=[
            pl.BlockSpec((M, BLK, K * OUT), lambda i: (0, i, 0)),
            pl.BlockSpec((1, 8, 128), lambda i: (i, 0, 0)),
        ],
        out_shape=[
            jax.ShapeDtypeStruct((M, N, K * OUT), jnp.float32),
            jax.ShapeDtypeStruct((NB, 8, 128), jnp.float32),
        ],
    )(outP, denP, bias, sem_W1, sem_b1, sem_W2)


def _final_body(z_ref, beta_ref, out_ref):
    acc = z_ref[0] * beta_ref[0:1, 0:1]
    acc = acc + z_ref[1] * beta_ref[0:1, 1:2]
    acc = acc + z_ref[2] * beta_ref[0:1, 2:3]
    out_ref[...] = acc


def _final_stage(z, beta_pad):
    return pl.pallas_call(
        _final_body,
        grid=(NB,),
        in_specs=[
            pl.BlockSpec((M, BLK, K * OUT), lambda i: (0, i, 0)),
            pl.BlockSpec((8, 128), lambda i: (0, 0)),
        ],
        out_specs=pl.BlockSpec((BLK, K * OUT), lambda i: (i, 0)),
        out_shape=jax.ShapeDtypeStruct((N, K * OUT), jnp.float32),
    )(z, beta_pad)


def kernel(h, edge_index_0, edge_index_1, edge_index_2, W, attn_l, attn_r,
           bias, sem_W1, sem_b1, sem_W2):
    rows = jnp.arange(K * OUT)
    Al = jnp.zeros((M, K * OUT, K), jnp.float32).at[
        :, rows, rows // OUT].set(attn_l.reshape(M, K * OUT))
    Ar = jnp.zeros((M, K * OUT, K), jnp.float32).at[
        :, rows, rows // OUT].set(attn_r.reshape(M, K * OUT))

    ft3, el3, er3 = _dense_stage(h, W, Al, Ar)

    outP, denPp = _sc_edge_stage(
        ft3[0], ft3[1], ft3[2], el3[0], el3[1], el3[2], er3[0], er3[1], er3[2],
        edge_index_0[0], edge_index_0[1], edge_index_1[0], edge_index_1[1],
        edge_index_2[0], edge_index_2[1])
    denP = denPp.reshape(M, NC, 1280 * 8, 16)[:, :, :N, :]

    z, wp = _combine_stage(outP, denP, bias, sem_W1, sem_b1, sem_W2)
    wsum = wp[:, 0, :M].sum(axis=0) / N
    beta = jax.nn.softmax(wsum)
    beta_pad = jnp.zeros((8, 128), jnp.float32).at[0, :M].set(beta)
    return _final_stage(z, beta_pad)
